# Initial kernel scaffold; baseline (speedup 1.0000x reference)
import jax
import jax.numpy as jnp
from jax.experimental import pallas as pl


def kernel(logits, decoded_tokens_list):
    raise NotImplementedError("write your pallas kernel here")


# speedup vs baseline: 11.0310x; 11.0310x over previous
import functools

import numpy as np
import jax
import jax.numpy as jnp
from jax import lax
from jax.experimental import pallas as pl
from jax.experimental.pallas import tpu as pltpu

B = 64
V = 100000
VP = 100352
NCHUNK = VP // 128
ROWS = 8
TOPK = 25
TOP_P = 0.8
WIN = 10
NEG = float(np.finfo(np.float32).min)
PAD_LOGIT = -1e30
TINY = float(np.finfo(np.float32).tiny)


def _u32(x):
    return int(np.uint32(x).astype(np.int32))


def _child_keys(seed):
    def rotl(x, d):
        return ((x << np.uint32(d)) | (x >> np.uint32(32 - d))).astype(np.uint32)

    k1 = np.uint32(seed >> 32)
    k2 = np.uint32(seed & 0xFFFFFFFF)
    ks = [k1, k2, np.uint32(k1 ^ k2 ^ np.uint32(0x1BD11BDA))]
    x0 = (np.zeros(3, np.uint32) + ks[0]).astype(np.uint32)
    x1 = (np.arange(3, dtype=np.uint32) + ks[1]).astype(np.uint32)
    rots = [[13, 15, 26, 6], [17, 29, 16, 24]]
    for r in range(5):
        for d in rots[r % 2]:
            x0 = (x0 + x1).astype(np.uint32)
            x1 = rotl(x1, d)
            x1 = (x1 ^ x0).astype(np.uint32)
        x0 = (x0 + ks[(r + 1) % 3]).astype(np.uint32)
        x1 = (x1 + ks[(r + 2) % 3] + np.uint32(r + 1)).astype(np.uint32)
    return np.stack([x0, x1], axis=1)


_KEYS = _child_keys(1234)
K_NUC = (_u32(_KEYS[0, 0]), _u32(_KEYS[0, 1]))
K_REP = (_u32(_KEYS[1, 0]), _u32(_KEYS[1, 1]))

_ROT0 = (13, 15, 26, 6)
_ROT1 = (17, 29, 16, 24)


def _threefry_bits(lo, key):
    k0, k1 = key
    k2 = _u32(np.int32(k0).view(np.uint32) ^ np.int32(k1).view(np.uint32)
              ^ np.uint32(0x1BD11BDA))
    ks = (k0, k1, k2)
    x0 = jnp.full_like(lo, k0)
    x1 = lo + k1

    def rotl(x, d):
        return lax.shift_left(x, d) | lax.shift_right_logical(x, 32 - d)

    for r in range(5):
        for d in (_ROT0 if r % 2 == 0 else _ROT1):
            x0 = x0 + x1
            x1 = rotl(x1, d)
            x1 = x1 ^ x0
        x0 = x0 + ks[(r + 1) % 3]
        x1 = x1 + ks[(r + 2) % 3] + (r + 1)
    return x0 ^ x1


def _gumbel(bits):
    fb = lax.shift_right_logical(bits, 9) | 0x3F800000
    f = lax.bitcast_convert_type(fb, jnp.float32) - 1.0
    u = jnp.maximum(TINY, f + TINY)
    return -jnp.log(-jnp.log(u))


def _body(x_ref, dec_ref, out_ref):
    g = pl.program_id(0)
    x = x_ref[...]

    m3 = jnp.max(x, axis=(1, 2), keepdims=True)
    s3 = jnp.sum(jnp.exp(x - m3), axis=(1, 2), keepdims=True)
    m = jnp.sum(m3, axis=1)
    s = jnp.sum(s3, axis=1)

    lane3 = lax.broadcasted_iota(jnp.int32, (ROWS, NCHUNK, 128), 2)
    lane13 = lax.broadcasted_iota(jnp.int32, (ROWS, 1, 128), 2)
    chunk3 = lax.broadcasted_iota(jnp.int32, (ROWS, NCHUNK, 1), 1)
    cm = jnp.max(x, axis=2, keepdims=True)
    cl = jnp.min(jnp.where(x == cm, lane3, 128), axis=2, keepdims=True)

    lane2 = lax.broadcasted_iota(jnp.int32, (ROWS, 128), 1)

    def step(k, carry):
        cm, cl, vals, idxs, cum, cums = carry
        v = jnp.max(cm, axis=(1, 2), keepdims=True)
        ci = jnp.min(jnp.where(cm == v, chunk3, NCHUNK), axis=(1, 2),
                     keepdims=True)
        oh = chunk3 == ci
        sel = jnp.sum(jnp.where(oh, x, 0.0), axis=1, keepdims=True)
        l = jnp.sum(jnp.where(oh, cl, 0), axis=(1, 2), keepdims=True)
        j2 = jnp.sum(ci * 128 + l, axis=1)
        v2 = jnp.sum(v, axis=1)
        pk = jnp.exp(v2 - m) / s
        cum = cum + pk
        at_k = lane2 == k
        vals = jnp.where(at_k, v2, vals)
        idxs = jnp.where(at_k, j2, idxs)
        cums = jnp.where(at_k, cum, cums)
        elig = (sel < v) | ((sel == v) & (lane13 > l))
        nv = jnp.max(jnp.where(elig, sel, NEG), axis=2, keepdims=True)
        nl = jnp.min(jnp.where(elig & (sel == nv), lane13, 128), axis=2,
                     keepdims=True)
        cm = jnp.where(oh, nv, cm)
        cl = jnp.where(oh, nl, cl)
        return cm, cl, vals, idxs, cum, cums

    init = (cm, cl,
            jnp.full((ROWS, 128), NEG, jnp.float32),
            jnp.zeros((ROWS, 128), jnp.int32),
            jnp.zeros((ROWS, 1), jnp.float32),
            jnp.zeros((ROWS, 128), jnp.float32))
    _, _, vals, idxs, _, cums = lax.fori_loop(0, TOPK, step, init)

    mask = (lane2 < TOPK) & ((cums <= TOP_P) | (lane2 == 0))
    p = jnp.exp(vals - m) / s
    sp = jnp.where(mask, p, 0.0)
    denom = jnp.sum(sp, axis=1, keepdims=True)
    spn = sp / denom

    rows2 = lax.broadcasted_iota(jnp.int32, (ROWS, 128), 0) + g * ROWS
    gn = _gumbel(_threefry_bits(rows2 * V + lane2, K_NUC))
    obj = jnp.where(mask, jnp.log(spn + 1e-30) + gn, NEG)
    amax = jnp.max(obj, axis=1, keepdims=True)
    pos = jnp.min(jnp.where(obj == amax, lane2, 128), axis=1, keepdims=True)
    top_id = jnp.sum(jnp.where(lane2 == pos, idxs, 0), axis=1, keepdims=True)

    dec = dec_ref[...]
    cnt = jnp.sum((dec == top_id).astype(jnp.int32), axis=1, keepdims=True)
    need = cnt >= 1
    out_ref[...] = jnp.broadcast_to(top_id, (ROWS, 128))

    @pl.when(jnp.any(need))
    def _resample():
        j3 = lax.broadcasted_iota(jnp.int32, (ROWS, NCHUNK, 128), 1) * 128 + lane3
        rows3 = lax.broadcasted_iota(jnp.int32, (ROWS, NCHUNK, 128), 0) + g * ROWS
        g3 = _gumbel(_threefry_bits(rows3 * V + j3, K_REP))
        obj3 = jnp.log(jnp.exp(x - m3) / s3 + 1e-30) + g3
        obj3 = jnp.where(j3 < V, obj3, NEG)
        rmax = jnp.max(obj3, axis=(1, 2), keepdims=True)
        rid3 = jnp.min(jnp.where(obj3 == rmax, j3, V), axis=(1, 2),
                       keepdims=True)
        rid = jnp.sum(rid3, axis=1)
        out_ref[...] = jnp.broadcast_to(jnp.where(need, rid, top_id), (ROWS, 128))


@jax.jit
def kernel(logits, decoded_tokens_list):
    xp = jnp.pad(logits, ((0, 0), (0, VP - V)), constant_values=PAD_LOGIT)
    xp = xp.reshape(B, NCHUNK, 128)
    dec = jnp.pad(decoded_tokens_list[:, -WIN:], ((0, 0), (0, 128 - WIN)),
                  constant_values=-1)
    out = pl.pallas_call(
        _body,
        grid=(B // ROWS,),
        in_specs=[
            pl.BlockSpec((ROWS, NCHUNK, 128), lambda g: (g, 0, 0)),
            pl.BlockSpec((ROWS, 128), lambda g: (g, 0)),
        ],
        out_specs=pl.BlockSpec((ROWS, 128), lambda g: (g, 0)),
        out_shape=jax.ShapeDtypeStruct((B, 128), jnp.int32),
        compiler_params=pltpu.CompilerParams(
            dimension_semantics=("arbitrary",),
        ),
    )(xp, dec)
    return out[:, 0]
